# baseline (speedup 1.0000x reference)
import functools

import jax
import jax.numpy as jnp
from jax import lax
from jax.experimental import pallas as pl
from jax.experimental.pallas import tpu as pltpu
from jax.experimental.pallas import tpu_sc as plsc

NUM_ROWS = 2600000
BATCH = 16384
N_FIELDS = 26

NC = 2
NS = 16
L = 16
NW = NC * NS
ROWS_W = BATCH // NW
IDX_W = ROWS_W * N_FIELDS
CHUNK = 128
NCHUNK = IDX_W // CHUNK

_mesh = plsc.VectorSubcoreMesh(core_axis_name="c", subcore_axis_name="s")

_SQ_BLK = 2600000
_SQ_GRID = -(-NUM_ROWS // _SQ_BLK)


def _tc_squeeze_body(w_ref, o_ref):
    o_ref[...] = w_ref[0, :]


_tc_squeeze = pl.pallas_call(
    _tc_squeeze_body,
    out_shape=jax.ShapeDtypeStruct((NUM_ROWS,), jnp.float32),
    grid=(_SQ_GRID,),
    in_specs=[pl.BlockSpec((1, _SQ_BLK), lambda i: (0, i))],
    out_specs=pl.BlockSpec((_SQ_BLK,), lambda i: (i,)),
)


@functools.partial(
    pl.kernel,
    out_type=jax.ShapeDtypeStruct((BATCH,), jnp.float32),
    mesh=_mesh,
    scratch_types=[
        pltpu.VMEM((IDX_W,), jnp.int32),
        pltpu.VMEM((IDX_W,), jnp.float32),
        pltpu.VMEM((ROWS_W,), jnp.float32),
        pltpu.VMEM((L,), jnp.float32),
        pltpu.SemaphoreType.DMA,
    ],
)
def _sc_embed_sum(idx_hbm, table_hbm, bias_hbm, out_hbm,
                  idx_v, vals_v, out_v, bias_v, sem):
    wid = lax.axis_index("s") * NC + lax.axis_index("c")
    base = wid * ROWS_W

    pltpu.sync_copy(bias_hbm, bias_v)
    pltpu.sync_copy(idx_hbm.at[wid], idx_v)

    @pl.loop(0, NCHUNK)
    def _fire(ch):
        pltpu.async_copy(table_hbm.at[idx_v.at[pl.ds(ch * CHUNK, CHUNK)]],
                         vals_v.at[pl.ds(ch * CHUNK, CHUNK)], sem)

    pltpu.make_async_copy(table_hbm.at[pl.ds(0, IDX_W)], vals_v, sem).wait()

    bias_vec = bias_v[...]

    @pl.loop(0, ROWS_W // L)
    def _reduce(j):
        acc = bias_vec
        for f in range(N_FIELDS):
            acc = acc + vals_v[pl.ds(f * ROWS_W + j * L, L)]
        out_v[pl.ds(j * L, L)] = acc

    pltpu.sync_copy(out_v, out_hbm.at[pl.ds(base, ROWS_W)])


def kernel(x, fc_weight, bias):
    idx = (x.astype(jnp.int32)
           .T.reshape(N_FIELDS, NW, ROWS_W)
           .transpose(1, 0, 2)
           .reshape(NW, IDX_W))
    table = _tc_squeeze(fc_weight.T)
    bias_b = jnp.broadcast_to(bias.astype(jnp.float32), (L,))
    out = _sc_embed_sum(idx, table, bias_b)
    return out.reshape(BATCH, 1)


# speedup vs baseline: 1.0315x; 1.0315x over previous
import functools

import jax
import jax.numpy as jnp
from jax import lax
from jax.experimental import pallas as pl
from jax.experimental.pallas import tpu as pltpu
from jax.experimental.pallas import tpu_sc as plsc

NUM_ROWS = 2600000
BATCH = 16384
N_FIELDS = 26

NC = 2
NS = 16
L = 16
NW = NC * NS
ROWS_W = BATCH // NW
IDX_W = ROWS_W * N_FIELDS
CHUNK = 128
NCHUNK = IDX_W // CHUNK

_mesh = plsc.VectorSubcoreMesh(core_axis_name="c", subcore_axis_name="s")

_SQ_BLK = 1310720
_SQ_GRID = -(-NUM_ROWS // _SQ_BLK)


def _tc_squeeze_body(w_ref, o_ref):
    o_ref[...] = w_ref[0, :]


_tc_squeeze = pl.pallas_call(
    _tc_squeeze_body,
    out_shape=jax.ShapeDtypeStruct((NUM_ROWS,), jnp.float32),
    grid=(_SQ_GRID,),
    in_specs=[pl.BlockSpec((1, _SQ_BLK), lambda i: (0, i))],
    out_specs=pl.BlockSpec((_SQ_BLK,), lambda i: (i,)),
)


@functools.partial(
    pl.kernel,
    out_type=jax.ShapeDtypeStruct((BATCH,), jnp.float32),
    mesh=_mesh,
    scratch_types=[
        pltpu.VMEM((IDX_W,), jnp.int32),
        pltpu.VMEM((IDX_W,), jnp.float32),
        pltpu.VMEM((ROWS_W,), jnp.float32),
        pltpu.VMEM((L,), jnp.float32),
        pltpu.SemaphoreType.DMA,
        pltpu.SemaphoreType.DMA,
    ],
)
def _sc_embed_sum(idx_hbm, table_hbm, bias_hbm, out_hbm,
                  idx_v, vals_v, out_v, bias_v, sem, sem_hi):
    wid = lax.axis_index("s") * NC + lax.axis_index("c")
    base = wid * ROWS_W

    pltpu.sync_copy(bias_hbm, bias_v)
    pltpu.sync_copy(idx_hbm.at[wid], idx_v)

    HF = N_FIELDS // 2
    HW = HF * ROWS_W
    NCH_H = HW // CHUNK

    @pl.loop(0, NCH_H)
    def _fire_lo(ch):
        pltpu.async_copy(table_hbm.at[idx_v.at[pl.ds(ch * CHUNK, CHUNK)]],
                         vals_v.at[pl.ds(ch * CHUNK, CHUNK)], sem)

    @pl.loop(NCH_H, NCHUNK)
    def _fire_hi(ch):
        pltpu.async_copy(table_hbm.at[idx_v.at[pl.ds(ch * CHUNK, CHUNK)]],
                         vals_v.at[pl.ds(ch * CHUNK, CHUNK)], sem_hi)

    pltpu.make_async_copy(table_hbm.at[pl.ds(0, HW)],
                          vals_v.at[pl.ds(0, HW)], sem).wait()

    bias_vec = bias_v[...]

    @pl.loop(0, ROWS_W // L)
    def _reduce_lo(j):
        acc = bias_vec
        for f in range(HF):
            acc = acc + vals_v[pl.ds(f * ROWS_W + j * L, L)]
        out_v[pl.ds(j * L, L)] = acc

    pltpu.make_async_copy(table_hbm.at[pl.ds(0, IDX_W - HW)],
                          vals_v.at[pl.ds(HW, IDX_W - HW)], sem_hi).wait()

    @pl.loop(0, ROWS_W // L)
    def _reduce_hi(j):
        acc = out_v[pl.ds(j * L, L)]
        for f in range(HF, N_FIELDS):
            acc = acc + vals_v[pl.ds(f * ROWS_W + j * L, L)]
        out_v[pl.ds(j * L, L)] = acc

    pltpu.sync_copy(out_v, out_hbm.at[pl.ds(base, ROWS_W)])


def kernel(x, fc_weight, bias):
    idx = (x.astype(jnp.int32)
           .T.reshape(N_FIELDS, NW, ROWS_W)
           .transpose(1, 0, 2)
           .reshape(NW, IDX_W))
    table = _tc_squeeze(fc_weight.T)
    bias_b = jnp.broadcast_to(bias.astype(jnp.float32), (L,))
    out = _sc_embed_sum(idx, table, bias_b)
    return out.reshape(BATCH, 1)


# speedup vs baseline: 1.0396x; 1.0079x over previous
import functools

import jax
import jax.numpy as jnp
from jax import lax
from jax.experimental import pallas as pl
from jax.experimental.pallas import tpu as pltpu
from jax.experimental.pallas import tpu_sc as plsc

NUM_ROWS = 2600000
BATCH = 16384
N_FIELDS = 26

NC = 2
NS = 16
L = 16
NW = NC * NS
ROWS_W = BATCH // NW
IDX_W = ROWS_W * N_FIELDS
CHUNK = 128
NCHUNK = IDX_W // CHUNK

_mesh = plsc.VectorSubcoreMesh(core_axis_name="c", subcore_axis_name="s")

_SQ_BLK = 1310720
_SQ_GRID = -(-NUM_ROWS // _SQ_BLK)


def _tc_squeeze_body(w_ref, o_ref):
    o_ref[...] = w_ref[0, :]


_tc_squeeze = pl.pallas_call(
    _tc_squeeze_body,
    out_shape=jax.ShapeDtypeStruct((NUM_ROWS,), jnp.float32),
    grid=(_SQ_GRID,),
    in_specs=[pl.BlockSpec((1, _SQ_BLK), lambda i: (0, i))],
    out_specs=pl.BlockSpec((_SQ_BLK,), lambda i: (i,)),
)


@functools.partial(
    pl.kernel,
    out_type=jax.ShapeDtypeStruct((BATCH,), jnp.float32),
    mesh=_mesh,
    scratch_types=[
        pltpu.VMEM((IDX_W,), jnp.int32),
        pltpu.VMEM((IDX_W,), jnp.float32),
        pltpu.VMEM((ROWS_W,), jnp.float32),
        pltpu.VMEM((L,), jnp.float32),
        pltpu.SemaphoreType.DMA,
        pltpu.SemaphoreType.DMA,
    ],
)
def _sc_embed_sum(idx_hbm, table_hbm, bias_hbm, out_hbm,
                  idx_v, vals_v, out_v, bias_v, sem, sem_hi):
    wid = lax.axis_index("s") * NC + lax.axis_index("c")
    base = wid * ROWS_W

    pltpu.sync_copy(bias_hbm, bias_v)
    pltpu.sync_copy(idx_hbm.at[wid], idx_v)

    HF = N_FIELDS // 2
    HW = HF * ROWS_W

    d_lo = pltpu.async_copy(table_hbm.at[idx_v.at[pl.ds(0, HW)]],
                            vals_v.at[pl.ds(0, HW)], sem)
    pltpu.async_copy(table_hbm.at[idx_v.at[pl.ds(HW, IDX_W - HW)]],
                     vals_v.at[pl.ds(HW, IDX_W - HW)], sem_hi)
    d_lo.wait()

    bias_vec = bias_v[...]

    @pl.loop(0, ROWS_W // L)
    def _reduce_lo(j):
        acc = bias_vec
        for f in range(HF):
            acc = acc + vals_v[pl.ds(f * ROWS_W + j * L, L)]
        out_v[pl.ds(j * L, L)] = acc

    pltpu.make_async_copy(table_hbm.at[pl.ds(0, IDX_W - HW)],
                          vals_v.at[pl.ds(HW, IDX_W - HW)], sem_hi).wait()

    @pl.loop(0, ROWS_W // L)
    def _reduce_hi(j):
        acc = out_v[pl.ds(j * L, L)]
        for f in range(HF, N_FIELDS):
            acc = acc + vals_v[pl.ds(f * ROWS_W + j * L, L)]
        out_v[pl.ds(j * L, L)] = acc

    pltpu.sync_copy(out_v, out_hbm.at[pl.ds(base, ROWS_W)])


def kernel(x, fc_weight, bias):
    idx = (x.astype(jnp.int32)
           .T.reshape(N_FIELDS, NW, ROWS_W)
           .transpose(1, 0, 2)
           .reshape(NW, IDX_W))
    table = _tc_squeeze(fc_weight.T)
    bias_b = jnp.broadcast_to(bias.astype(jnp.float32), (L,))
    out = _sc_embed_sum(idx, table, bias_b)
    return out.reshape(BATCH, 1)
